# baseline (speedup 1.0000x reference)
import functools

import jax
import jax.numpy as jnp
from jax import lax
from jax.experimental import pallas as pl
from jax.experimental.pallas import tpu as pltpu
from jax.experimental.pallas import tpu_sc as plsc


def _tc_reduce_body(x_ref, attr_ref, a_ref, *, num_patches, patch, m_blk, mm):
    i = pl.program_id(0)
    for j in range(m_blk):
        b = (i * m_blk + j) // mm
        s = jnp.sign(x_ref[b])
        v = jnp.maximum(attr_ref[j] * s[None, :], 0.0)
        psum = v.reshape(num_patches, patch, v.shape[-1]).sum(axis=(1, 2))
        a_ref[j, 0] = psum


def _sc_pd_body(x_hbm, wt_hbm, out_hbm, xv, wv, outv, *, mm, num_patches,
                patch):
    nc = 2
    wid = lax.axis_index("s") * nc + lax.axis_index("c")
    pltpu.sync_copy(x_hbm, xv)
    pltpu.sync_copy(wt_hbm, wv)
    for j in range(2):
        bm = wid * 2 + j
        b = bm // mm
        m = bm % mm
        for p in range(num_patches):
            acc = jnp.zeros((16,), jnp.float32)
            for c in range(patch // 16):
                off = p * patch + c * 16
                acc = acc + xv[b, pl.ds(off, 16)] * wv[m, pl.ds(off, 16)]
            outv[pl.ds((j * num_patches + p) * 16, 16)] = acc
    sz = 2 * num_patches * 16
    pltpu.sync_copy(outv, out_hbm.at[pl.ds(wid * sz, sz)])


def _tail_body(a_ref, pd_sc_ref, biasr_ref, out_ref, *, num_patches, patch):
    P = num_patches
    a_full = a_ref[:]
    pd = jnp.sum(pd_sc_ref[:], axis=-1)

    a2 = a_full
    ap = a2[:, :, None]
    aq = a2[:, None, :]
    qi = jax.lax.broadcasted_iota(jnp.int32, (a2.shape[0], P, P), 2)
    pi = jax.lax.broadcasted_iota(jnp.int32, (a2.shape[0], P, P), 1)
    beats = (aq > ap) | ((aq == ap) & (qi < pi))
    rank = jnp.sum(beats.astype(jnp.float32), axis=-1)
    wgt = jnp.float32(P) - rank

    S = jnp.sum(wgt * pd, axis=-1, keepdims=True)
    biasr = biasr_ref[:]
    inf0 = jnp.sum(pd, axis=-1, keepdims=True) + biasr
    dx = jnp.float32(1.0 / (P + 2))
    out_ref[:] = dx * (0.5 * (1.0 + biasr / inf0)
                       + (jnp.float32(P) * inf0 - S) / inf0)


def kernel(x, attr, mask, W, bias):
    B, M, F, N = attr.shape
    patch = int(F * 0.0625)
    P = F // patch
    BM = B * M

    attr3 = attr.reshape(BM, F, N)

    M_BLK = 2
    a = pl.pallas_call(
        functools.partial(_tc_reduce_body, num_patches=P, patch=patch,
                          m_blk=M_BLK, mm=M),
        grid=(BM // M_BLK,),
        in_specs=[
            pl.BlockSpec((B, N), lambda i: (0, 0)),
            pl.BlockSpec((M_BLK, F, N), lambda i: (i, 0, 0)),
        ],
        out_specs=pl.BlockSpec((M_BLK, 1, P), lambda i: (i, 0, 0)),
        out_shape=jax.ShapeDtypeStruct((BM, 1, P), jnp.float32),
    )(x, attr3)

    a2 = a.reshape(BM, P)

    mesh = plsc.VectorSubcoreMesh(core_axis_name="c", subcore_axis_name="s")
    pd_sc = pl.kernel(
        functools.partial(_sc_pd_body, mm=M, num_patches=P, patch=patch),
        out_type=jax.ShapeDtypeStruct((BM * P * 16,), jnp.float32),
        mesh=mesh,
        scratch_types=[
            pltpu.VMEM((B, N), jnp.float32),
            pltpu.VMEM((M, N), jnp.float32),
            pltpu.VMEM((2 * P * 16,), jnp.float32),
        ],
    )(x, W.T)

    biasr = jnp.tile(bias, B).reshape(BM, 1)

    out_flat = pl.pallas_call(
        functools.partial(_tail_body, num_patches=P, patch=patch),
        out_shape=jax.ShapeDtypeStruct((BM, 1), jnp.float32),
    )(a2, pd_sc.reshape(BM, P, 16), biasr)
    return out_flat.reshape(B, M)


# speedup vs baseline: 1.2396x; 1.2396x over previous
import functools

import jax
import jax.numpy as jnp
from jax.experimental import pallas as pl


def _tc_reduce_body(x_ref, attr_ref, a_ref, *, num_patches, patch, m_blk, mm):
    i = pl.program_id(0)
    for j in range(m_blk):
        b = (i * m_blk + j) // mm
        s = jnp.sign(x_ref[b])
        v = jnp.maximum(attr_ref[j] * s[None, :], 0.0)
        psum = v.reshape(num_patches, patch, v.shape[-1]).sum(axis=(1, 2))
        a_ref[j, 0] = psum


def _tail_body(a_ref, xr_ref, wt_ref, biasr_ref, out_ref, *, num_patches,
               patch):
    P = num_patches
    a_full = a_ref[:]
    T = xr_ref[:] * wt_ref[:]
    N = T.shape[-1]
    n_iota = jax.lax.broadcasted_iota(jnp.int32, (N, P), 0)
    p_iota = jax.lax.broadcasted_iota(jnp.int32, (N, P), 1)
    ind = ((n_iota // patch) == p_iota).astype(jnp.float32)
    pd = jnp.dot(T, ind, preferred_element_type=jnp.float32)

    a2 = a_full
    ap = a2[:, :, None]
    aq = a2[:, None, :]
    qi = jax.lax.broadcasted_iota(jnp.int32, (a2.shape[0], P, P), 2)
    pi = jax.lax.broadcasted_iota(jnp.int32, (a2.shape[0], P, P), 1)
    beats = (aq > ap) | ((aq == ap) & (qi < pi))
    rank = jnp.sum(beats.astype(jnp.float32), axis=-1)
    wgt = jnp.float32(P) - rank

    S = jnp.sum(wgt * pd, axis=-1, keepdims=True)
    biasr = biasr_ref[:]
    inf0 = jnp.sum(pd, axis=-1, keepdims=True) + biasr
    dx = jnp.float32(1.0 / (P + 2))
    out_ref[:] = dx * (0.5 * (1.0 + biasr / inf0)
                       + (jnp.float32(P) * inf0 - S) / inf0)


def kernel(x, attr, mask, W, bias):
    B, M, F, N = attr.shape
    patch = int(F * 0.0625)
    P = F // patch
    BM = B * M

    attr3 = attr.reshape(BM, F, N)

    M_BLK = 2
    a = pl.pallas_call(
        functools.partial(_tc_reduce_body, num_patches=P, patch=patch,
                          m_blk=M_BLK, mm=M),
        grid=(BM // M_BLK,),
        in_specs=[
            pl.BlockSpec((B, N), lambda i: (0, 0)),
            pl.BlockSpec((M_BLK, F, N), lambda i: (i, 0, 0)),
        ],
        out_specs=pl.BlockSpec((M_BLK, 1, P), lambda i: (i, 0, 0)),
        out_shape=jax.ShapeDtypeStruct((BM, 1, P), jnp.float32),
    )(x, attr3)

    a2 = a.reshape(BM, P)

    xr = jnp.repeat(x, M, axis=0)
    wt = jnp.tile(W.T, (B, 1))
    biasr = jnp.tile(bias, B).reshape(BM, 1)

    out_flat = pl.pallas_call(
        functools.partial(_tail_body, num_patches=P, patch=patch),
        out_shape=jax.ShapeDtypeStruct((BM, 1), jnp.float32),
    )(a2, xr, wt, biasr)
    return out_flat.reshape(B, M)
